# Initial kernel scaffold; baseline (speedup 1.0000x reference)
import functools

import jax
import jax.numpy as jnp
from jax import lax
from jax.experimental import pallas as pl
from jax.experimental.pallas import tpu as pltpu
from jax.experimental.pallas import tpu_sc as plsc

_NC = 2
_NS = 16
_NW = _NC * _NS

_IL = 128
_KG = 2
_CHUNK = _IL * _KG


@functools.lru_cache(maxsize=None)
def _make_gather(N, V, D):
    per_w = N // _NW
    n_chunks = per_w // _CHUNK
    assert per_w % _CHUNK == 0

    mesh = plsc.VectorSubcoreMesh(core_axis_name="c", subcore_axis_name="s")

    @functools.partial(
        pl.kernel,
        out_type=jax.ShapeDtypeStruct((N, D), jnp.float32),
        mesh=mesh,
        scratch_types=[
            pltpu.VMEM((_KG, _IL), jnp.int32),
            pltpu.VMEM((_CHUNK, D), jnp.float32),
            pltpu.SemaphoreType.DMA,
        ],
    )
    def gather_kernel(ids_hbm, table_hbm, out_hbm, idx_v, rows_v, gsem):
        wid = lax.axis_index("s") * _NC + lax.axis_index("c")
        base = wid * per_w

        def body(g, _):
            off = base + g * _CHUNK
            pltpu.sync_copy(ids_hbm.at[pl.ds(off // _IL, _KG)], idx_v)
            for j in range(_KG):
                pltpu.async_copy(
                    table_hbm.at[idx_v.at[j]],
                    rows_v.at[pl.ds(j * _IL, _IL)],
                    gsem,
                )
            for j in range(_KG):
                pltpu.make_async_copy(
                    table_hbm.at[idx_v.at[j]],
                    rows_v.at[pl.ds(j * _IL, _IL)],
                    gsem,
                ).wait()
            pltpu.sync_copy(rows_v, out_hbm.at[pl.ds(off, _CHUNK)])
            return 0

        lax.fori_loop(0, n_chunks, body, 0)

    return gather_kernel


def kernel(input_ids, table):
    B, S = input_ids.shape
    V, D = table.shape
    N = B * S
    ids_t = jnp.transpose(input_ids).reshape(N // _IL, _IL)
    out_flat = _make_gather(N, V, D)(ids_t, table)
    return out_flat.reshape(S, B, D)


# speedup vs baseline: 6.9156x; 6.9156x over previous
import functools

import jax
import jax.numpy as jnp
from jax import lax
from jax.experimental import pallas as pl
from jax.experimental.pallas import tpu as pltpu
from jax.experimental.pallas import tpu_sc as plsc

_NC = 2
_NS = 16
_NW = _NC * _NS

_IL = 128
_KG = 2
_CHUNK = _IL * _KG


@functools.lru_cache(maxsize=None)
def _make_gather(N, V, D):
    per_w = N // _NW
    n_chunks = per_w // _CHUNK
    assert per_w % _CHUNK == 0

    mesh = plsc.VectorSubcoreMesh(core_axis_name="c", subcore_axis_name="s")

    @functools.partial(
        pl.kernel,
        out_type=jax.ShapeDtypeStruct((N, D), jnp.float32),
        mesh=mesh,
        scratch_types=[
            pltpu.VMEM((_CHUNK,), jnp.int32),
            pltpu.VMEM((_CHUNK, D), jnp.float32),
            pltpu.SemaphoreType.DMA,
        ],
    )
    def gather_kernel(ids_hbm, table_hbm, out_hbm, idx_v, rows_v, gsem):
        wid = lax.axis_index("s") * _NC + lax.axis_index("c")
        base = wid * per_w

        def body(g, _):
            off = base + g * _CHUNK
            pltpu.sync_copy(ids_hbm.at[pl.ds(off, _CHUNK)], idx_v)
            for j in range(_KG):
                pltpu.async_copy(
                    table_hbm.at[idx_v.at[pl.ds(j * _IL, _IL)]],
                    rows_v.at[pl.ds(j * _IL, _IL)],
                    gsem,
                )
            for j in range(_KG):
                pltpu.make_async_copy(
                    table_hbm.at[idx_v.at[pl.ds(j * _IL, _IL)]],
                    rows_v.at[pl.ds(j * _IL, _IL)],
                    gsem,
                ).wait()
            pltpu.sync_copy(rows_v, out_hbm.at[pl.ds(off, _CHUNK)])
            return 0

        lax.fori_loop(0, n_chunks, body, 0)

    return gather_kernel


def kernel(input_ids, table):
    B, S = input_ids.shape
    V, D = table.shape
    N = B * S
    ids_t = jnp.transpose(input_ids).reshape(N)
    out_flat = _make_gather(N, V, D)(ids_t, table)
    return out_flat.reshape(S, B, D)


# speedup vs baseline: 9.1668x; 1.3255x over previous
import functools

import jax
import jax.numpy as jnp
from jax import lax
from jax.experimental import pallas as pl
from jax.experimental.pallas import tpu as pltpu
from jax.experimental.pallas import tpu_sc as plsc

_NC = 2
_NS = 16
_NW = _NC * _NS

_IL = 128
_KG = 2
_CHUNK = _IL * _KG


@functools.lru_cache(maxsize=None)
def _make_gather(N, V, D):
    per_w = n_rows = N // _NW
    n_chunks = per_w // _CHUNK
    assert per_w % _CHUNK == 0 and n_chunks % 2 == 0
    n_pairs = n_chunks // 2

    mesh = plsc.VectorSubcoreMesh(core_axis_name="c", subcore_axis_name="s")

    @functools.partial(
        pl.kernel,
        out_type=jax.ShapeDtypeStruct((N, D), jnp.float32),
        mesh=mesh,
        scratch_types=[
            pltpu.VMEM((per_w,), jnp.int32),
            pltpu.VMEM((2, _CHUNK, D), jnp.float32),
            pltpu.SemaphoreType.DMA,
            pltpu.SemaphoreType.DMA,
            pltpu.SemaphoreType.DMA,
            pltpu.SemaphoreType.DMA,
        ],
    )
    def gather_kernel(ids_hbm, table_hbm, out_hbm, idx_v, rows_v,
                      gsem0, gsem1, ssem0, ssem1):
        wid = lax.axis_index("s") * _NC + lax.axis_index("c")
        base = wid * per_w
        gsems = (gsem0, gsem1)
        ssems = (ssem0, ssem1)

        pltpu.sync_copy(ids_hbm.at[pl.ds(base, per_w)], idx_v)

        def gather_issue(g, slot):
            for j in range(_KG):
                pltpu.async_copy(
                    table_hbm.at[idx_v.at[pl.ds(g * _CHUNK + j * _IL, _IL)]],
                    rows_v.at[slot, pl.ds(j * _IL, _IL)],
                    gsems[slot],
                )

        def gather_wait(g, slot):
            for j in range(_KG):
                pltpu.make_async_copy(
                    table_hbm.at[idx_v.at[pl.ds(g * _CHUNK + j * _IL, _IL)]],
                    rows_v.at[slot, pl.ds(j * _IL, _IL)],
                    gsems[slot],
                ).wait()

        def store_issue(g, slot):
            pltpu.async_copy(
                rows_v.at[slot],
                out_hbm.at[pl.ds(base + g * _CHUNK, _CHUNK)],
                ssems[slot],
            )

        def store_wait(g, slot):
            pltpu.make_async_copy(
                rows_v.at[slot],
                out_hbm.at[pl.ds(base + g * _CHUNK, _CHUNK)],
                ssems[slot],
            ).wait()

        gather_issue(0, 0)

        def body(p, _):
            g0 = 2 * p
            g1 = g0 + 1

            @pl.when(p >= 1)
            def _():
                store_wait(g0 - 1, 1)

            gather_issue(g1, 1)
            gather_wait(g0, 0)
            store_issue(g0, 0)
            gather_wait(g1, 1)
            store_issue(g1, 1)
            store_wait(g0, 0)

            @pl.when(p + 1 < n_pairs)
            def _():
                gather_issue(g0 + 2, 0)

            return 0

        lax.fori_loop(0, n_pairs, body, 0)
        store_wait(n_chunks - 1, 1)

    return gather_kernel


def kernel(input_ids, table):
    B, S = input_ids.shape
    V, D = table.shape
    N = B * S
    ids_t = jnp.transpose(input_ids).reshape(N)
    out_flat = _make_gather(N, V, D)(ids_t, table)
    return out_flat.reshape(S, B, D)


# speedup vs baseline: 9.3610x; 1.0212x over previous
import functools

import jax
import jax.numpy as jnp
from jax import lax
from jax.experimental import pallas as pl
from jax.experimental.pallas import tpu as pltpu
from jax.experimental.pallas import tpu_sc as plsc

_NC = 2
_NS = 16
_NW = _NC * _NS

_IL = 128
_KG = 2
_CHUNK = _IL * _KG
_NBUF = 3


@functools.lru_cache(maxsize=None)
def _make_gather(N, V, D):
    per_w = N // _NW
    n = per_w // _CHUNK
    assert per_w % _CHUNK == 0 and (n - 4) % _NBUF == 0 and n >= 2 * _NBUF

    mesh = plsc.VectorSubcoreMesh(core_axis_name="c", subcore_axis_name="s")

    @functools.partial(
        pl.kernel,
        out_type=jax.ShapeDtypeStruct((N, D), jnp.float32),
        mesh=mesh,
        scratch_types=[
            pltpu.VMEM((per_w,), jnp.int32),
            pltpu.VMEM((_NBUF, _CHUNK, D), jnp.float32),
            [pltpu.SemaphoreType.DMA] * _NBUF,
            [pltpu.SemaphoreType.DMA] * _NBUF,
        ],
    )
    def gather_kernel(ids_hbm, table_hbm, out_hbm, idx_v, rows_v,
                      gsems, ssems):
        wid = lax.axis_index("s") * _NC + lax.axis_index("c")
        base = wid * per_w

        pltpu.sync_copy(ids_hbm.at[pl.ds(base, per_w)], idx_v)

        def gather_issue(g, slot):
            for j in range(_KG):
                pltpu.async_copy(
                    table_hbm.at[idx_v.at[pl.ds(g * _CHUNK + j * _IL, _IL)]],
                    rows_v.at[slot, pl.ds(j * _IL, _IL)],
                    gsems[slot],
                )

        def gather_wait(g, slot):
            for j in range(_KG):
                pltpu.make_async_copy(
                    table_hbm.at[idx_v.at[pl.ds(g * _CHUNK + j * _IL, _IL)]],
                    rows_v.at[slot, pl.ds(j * _IL, _IL)],
                    gsems[slot],
                ).wait()

        def store_issue(g, slot):
            pltpu.async_copy(
                rows_v.at[slot],
                out_hbm.at[pl.ds(base + g * _CHUNK, _CHUNK)],
                ssems[slot],
            )

        def store_wait(g, slot):
            pltpu.make_async_copy(
                rows_v.at[slot],
                out_hbm.at[pl.ds(base + g * _CHUNK, _CHUNK)],
                ssems[slot],
            ).wait()

        gather_issue(0, 0)
        gather_issue(1, 1)
        gather_wait(0, 0)
        store_issue(0, 0)
        gather_issue(2, 2)
        gather_wait(1, 1)
        store_issue(1, 1)

        def body(q, _):
            for j in range(_NBUF):
                i = _NBUF * q + 2 + j
                slot = (2 + j) % _NBUF
                store_wait(i - 2, (slot + 1) % _NBUF)
                gather_issue(i + 1, (slot + 1) % _NBUF)
                gather_wait(i, slot)
                store_issue(i, slot)
            return 0

        lax.fori_loop(0, (n - 2 - 2) // _NBUF, body, 0)

        for i in (n - 2, n - 1):
            slot = i % _NBUF
            store_wait(i - 2, (slot + 1) % _NBUF)
            if i + 1 < n:
                gather_issue(i + 1, (slot + 1) % _NBUF)
            gather_wait(i, slot)
            store_issue(i, slot)
        store_wait(n - 2, (n - 2) % _NBUF)
        store_wait(n - 1, (n - 1) % _NBUF)

    return gather_kernel


def kernel(input_ids, table):
    B, S = input_ids.shape
    V, D = table.shape
    N = B * S
    ids_t = jnp.transpose(input_ids).reshape(N)
    out_flat = _make_gather(N, V, D)(ids_t, table)
    return out_flat.reshape(S, B, D)
